# baseline (speedup 1.0000x reference)
import jax
import jax.numpy as jnp
from jax import lax
from jax.experimental import pallas as pl
from jax.experimental.pallas import tpu as pltpu
from jax.experimental.pallas import tpu_sc as plsc

BATCH = 16384
EMBED = 64
NC = 2
NS = 16
NW = NC * NS
B_PER_W = BATCH // NW
CHUNK = 128
NCHUNK = B_PER_W // CHUNK


def _sc_kernel(uids_hbm, pids_hbm, user_t2, product_t2, wb_hbm, out_hbm,
               uidx, pidx, uphys, pphys, ubuf, pbuf, wv, outv, sems):
    wid = lax.axis_index("s") * NC + lax.axis_index("c")
    base = wid * B_PER_W

    pltpu.sync_copy(uids_hbm.at[pl.ds(base, B_PER_W)], uidx)
    pltpu.sync_copy(pids_hbm.at[pl.ds(base, B_PER_W)], pidx)
    pltpu.sync_copy(wb_hbm, wv)

    for k in range(B_PER_W // 16):
        sl = pl.ds(k * 16, 16)
        uphys[sl] = jax.lax.shift_right_logical(uidx[sl], 1)
        pphys[sl] = jax.lax.shift_right_logical(pidx[sl], 1)

    def fire(j):
        buf = j % 2
        return (
            pltpu.async_copy(
                user_t2.at[uphys.at[pl.ds(j * CHUNK, CHUNK)]], ubuf.at[buf],
                sems.at[buf]),
            pltpu.async_copy(
                product_t2.at[pphys.at[pl.ds(j * CHUNK, CHUNK)]], pbuf.at[buf],
                sems.at[buf]),
        )

    lane = lax.iota(jnp.int32, 16)
    wchunks = [wv[pl.ds(c * 16, 16)] for c in range(EMBED // 16)]
    bias = wv[pl.ds(EMBED, 16)][0]

    inflight = fire(0)
    for j in range(NCHUNK):
        for c in inflight:
            c.wait()
        if j + 1 < NCHUNK:
            inflight = fire(j + 1)
        buf = j % 2
        urows = ubuf.at[buf]
        prows = pbuf.at[buf]

        def body(g, carry):
            k = j * CHUNK + g * 16
            rows = lane + g * 16
            ucol0 = (uidx[pl.ds(k, 16)] & 1) * EMBED
            pcol0 = (pidx[pl.ds(k, 16)] & 1) * EMBED
            acc = jnp.zeros((16,), jnp.float32)
            for e in range(EMBED):
                u = plsc.load_gather(urows, [rows, ucol0 + e])
                p = plsc.load_gather(prows, [rows, pcol0 + e])
                acc = acc + u * p * wchunks[e // 16][e % 16]
            outv[pl.ds(k, 16)] = acc + bias
            return carry

        lax.fori_loop(0, CHUNK // 16, body, 0)

    pltpu.sync_copy(outv, out_hbm.at[pl.ds(base, B_PER_W)])


@jax.jit
def kernel(user_ids, product_ids, user_table, product_table, fc_w, fc_b):
    nu, np_ = user_table.shape[0], product_table.shape[0]
    user_t2 = user_table.reshape(nu // 2, 2 * EMBED)
    product_t2 = product_table.reshape(np_ // 2, 2 * EMBED)
    uids = user_ids.astype(jnp.int32)
    pids = product_ids.astype(jnp.int32)
    wb = jnp.zeros((128,), jnp.float32)
    wb = wb.at[:EMBED].set(fc_w[0]).at[EMBED].set(fc_b[0])

    mesh = plsc.VectorSubcoreMesh(core_axis_name="c", subcore_axis_name="s")
    run = pl.kernel(
        _sc_kernel,
        out_type=jax.ShapeDtypeStruct((BATCH,), jnp.float32),
        mesh=mesh,
        compiler_params=pltpu.CompilerParams(needs_layout_passes=False),
        scratch_types=[
            pltpu.VMEM((B_PER_W,), jnp.int32),
            pltpu.VMEM((B_PER_W,), jnp.int32),
            pltpu.VMEM((B_PER_W,), jnp.int32),
            pltpu.VMEM((B_PER_W,), jnp.int32),
            pltpu.VMEM((2, CHUNK, 2 * EMBED), jnp.float32),
            pltpu.VMEM((2, CHUNK, 2 * EMBED), jnp.float32),
            pltpu.VMEM((128,), jnp.float32),
            pltpu.VMEM((B_PER_W,), jnp.float32),
            pltpu.SemaphoreType.DMA((2,)),
        ],
    )
    return run(uids, pids, user_t2, product_t2, wb)


# speedup vs baseline: 2.1932x; 2.1932x over previous
import jax
import jax.numpy as jnp
from jax import lax
from jax.experimental import pallas as pl
from jax.experimental.pallas import tpu as pltpu
from jax.experimental.pallas import tpu_sc as plsc

BATCH = 16384
EMBED = 64
NC = 2
NS = 16
NW = NC * NS
B_PER_W = BATCH // NW
CHUNK = 128
NCHUNK = B_PER_W // CHUNK
SENTINEL = 0x7FFFFF


def _k1_user(su_hbm, ut3, wb_hbm, u1d_hbm,
             suv, colv, sbuf, ustage, wv, sems):
    wid = lax.axis_index("s") * NC + lax.axis_index("c")
    base = wid * B_PER_W

    pltpu.sync_copy(su_hbm.at[pl.ds(base, B_PER_W)], suv.at[pl.ds(0, B_PER_W)])
    pltpu.sync_copy(wb_hbm, wv)

    for k in range(B_PER_W // 16):
        colv[pl.ds(k * 16, 16)] = jax.lax.shift_right_logical(
            suv[pl.ds(k * 16, 16)], 7)
    colv[pl.ds(B_PER_W, 16)] = jnp.full((16,), SENTINEL, jnp.int32)

    c_lo = colv[pl.ds(0, 16)][0]
    c_hi = colv[pl.ds(B_PER_W - 16, 16)][15]

    wchunks = [wv[pl.ds(q * 16, 16)] for q in range(EMBED // 16)]
    iot = lax.iota(jnp.int32, 16)
    ehq = [(iot + 16 * q) >> 3 for q in range(4)]
    elq = [(iot + 16 * q) & 7 for q in range(4)]
    eoffq = [iot * 16 + 256 * q for q in range(4)]

    def fire(c, b):
        @pl.when(c <= c_hi)
        def _():
            pltpu.async_copy(
                ut3.at[:, :, pl.ds(c * CHUNK, CHUNK)], sbuf.at[b], sems.at[b])

    def drain(c, b):
        @pl.when(c <= c_hi)
        def _():
            pltpu.make_async_copy(
                ut3.at[:, :, pl.ds(c * CHUNK, CHUNK)], sbuf.at[b],
                sems.at[b]).wait()

    def extract(c, b, carry):
        def cond(carry):
            jp, col_j = carry
            return jnp.logical_and(jp < B_PER_W, col_j == c)

        def wbody(carry):
            jp, _ = carry
            jpv = jnp.full((16,), jp, jnp.int32)
            su_j = plsc.load_gather(suv, [jpv])
            liv = su_j & 127
            sbase = ((jpv >> 4) << 10) + (jpv & 15)
            for q in range(4):
                val = plsc.load_gather(sbuf, [jnp.full((16,), b, jnp.int32),
                                              ehq[q], elq[q], liv])
                plsc.store_scatter(ustage, [sbase + eoffq[q]],
                                   val * wchunks[q])
            jp1 = jp + 1
            col1 = plsc.load_gather(colv, [jnp.full((16,), jp1, jnp.int32)])[0]
            return (jp1, col1)

        return lax.while_loop(cond, wbody, carry)

    fire(c_lo, 0)
    nsteps = jax.lax.shift_right_logical(c_hi - c_lo, 1) + 1

    def pair_body(t, carry):
        c0 = c_lo + 2 * t
        fire(c0 + 1, 1)
        drain(c0, 0)
        carry = extract(c0, 0, carry)
        fire(c0 + 2, 0)
        drain(c0 + 1, 1)
        carry = extract(c0 + 1, 1, carry)
        return carry

    col0 = colv[pl.ds(0, 16)][0]
    lax.fori_loop(0, nsteps, pair_body, (jnp.int32(0), col0))

    pltpu.sync_copy(ustage, u1d_hbm.at[pl.ds(base * EMBED, B_PER_W * EMBED)])


def _k2_join(u1d_hbm, pids_hbm, product_t2, wb_hbm, out_hbm,
             uslab, pidx, pphys, pbuf, wv, outv, sems):
    wid = lax.axis_index("s") * NC + lax.axis_index("c")
    base = wid * B_PER_W

    pltpu.sync_copy(u1d_hbm.at[pl.ds(base * EMBED, B_PER_W * EMBED)], uslab)
    pltpu.sync_copy(pids_hbm.at[pl.ds(base, B_PER_W)], pidx)
    pltpu.sync_copy(wb_hbm, wv)

    for k in range(B_PER_W // 16):
        sl = pl.ds(k * 16, 16)
        pphys[sl] = jax.lax.shift_right_logical(pidx[sl], 1)

    def fire(j):
        b = j % 2
        return pltpu.async_copy(
            product_t2.at[pphys.at[pl.ds(j * CHUNK, CHUNK)]], pbuf.at[b],
            sems.at[b])

    lane = lax.iota(jnp.int32, 16)
    bias = wv[pl.ds(EMBED, 16)][0]

    inflight = fire(0)
    for j in range(NCHUNK):
        inflight.wait()
        if j + 1 < NCHUNK:
            inflight = fire(j + 1)
        prows = pbuf.at[j % 2]

        def body(g, carry):
            k = j * CHUNK + g * 16
            rows = lane + g * 16
            pcol0 = (pidx[pl.ds(k, 16)] & 1) * EMBED
            acc = jnp.zeros((16,), jnp.float32)
            ub = j * CHUNK * EMBED + g * 16 * EMBED
            for e in range(EMBED):
                u = uslab[pl.ds(ub + e * 16, 16)]
                p = plsc.load_gather(prows, [rows, pcol0 + e])
                acc = acc + u * p
            outv[pl.ds(k, 16)] = acc + bias
            return carry

        lax.fori_loop(0, CHUNK // 16, body, 0)

    pltpu.sync_copy(outv, out_hbm.at[pl.ds(base, B_PER_W)])


@jax.jit
def kernel(user_ids, product_ids, user_table, product_table, fc_w, fc_b):
    uids = user_ids.astype(jnp.int32)
    pids = product_ids.astype(jnp.int32)
    order0 = lax.iota(jnp.int32, BATCH)
    su, order = lax.sort_key_val(uids, order0)
    pids_s = jnp.take(pids, order)

    ut3 = user_table.T.reshape(8, 8, user_table.shape[0])
    product_t2 = product_table.reshape(product_table.shape[0] // 2, 2 * EMBED)

    wb = jnp.zeros((128,), jnp.float32)
    wb = wb.at[:EMBED].set(fc_w[0]).at[EMBED].set(fc_b[0])

    mesh = plsc.VectorSubcoreMesh(core_axis_name="c", subcore_axis_name="s")
    params = pltpu.CompilerParams(needs_layout_passes=False)

    k1 = pl.kernel(
        _k1_user,
        out_type=jax.ShapeDtypeStruct((BATCH * EMBED,), jnp.float32),
        mesh=mesh, compiler_params=params,
        scratch_types=[
            pltpu.VMEM((B_PER_W,), jnp.int32),
            pltpu.VMEM((B_PER_W + 16,), jnp.int32),
            pltpu.VMEM((2, 8, 8, CHUNK), jnp.float32),
            pltpu.VMEM((B_PER_W * EMBED,), jnp.float32),
            pltpu.VMEM((128,), jnp.float32),
            pltpu.SemaphoreType.DMA((2,)),
        ],
    )
    u1d = k1(su, ut3, wb)

    k2 = pl.kernel(
        _k2_join,
        out_type=jax.ShapeDtypeStruct((BATCH,), jnp.float32),
        mesh=mesh, compiler_params=params,
        scratch_types=[
            pltpu.VMEM((B_PER_W * EMBED,), jnp.float32),
            pltpu.VMEM((B_PER_W,), jnp.int32),
            pltpu.VMEM((B_PER_W,), jnp.int32),
            pltpu.VMEM((2, CHUNK, 2 * EMBED), jnp.float32),
            pltpu.VMEM((128,), jnp.float32),
            pltpu.VMEM((B_PER_W,), jnp.float32),
            pltpu.SemaphoreType.DMA((2,)),
        ],
    )
    out_s = k2(u1d, pids_s, product_t2, wb)

    return jnp.zeros((BATCH,), jnp.float32).at[order].set(out_s)


# speedup vs baseline: 3.2657x; 1.4890x over previous
import jax
import jax.numpy as jnp
from jax import lax
from jax.experimental import pallas as pl
from jax.experimental.pallas import tpu as pltpu
from jax.experimental.pallas import tpu_sc as plsc

BATCH = 16384
EMBED = 64
NC = 2
NS = 16
NW = NC * NS
B_PER_W = BATCH // NW
CHUNK = 128
NCHUNK = B_PER_W // CHUNK
RING = 4
SENTINEL = 0x7FFFFF


def _k1_user(su_hbm, ut3, wb_hbm, u1d_hbm, suv, colv, sbuf, ustage, wv, sems):
    wid = lax.axis_index("s") * NC + lax.axis_index("c")
    base = wid * B_PER_W

    pltpu.sync_copy(su_hbm.at[pl.ds(base, B_PER_W)], suv)
    pltpu.sync_copy(wb_hbm, wv)

    for k in range(B_PER_W // 16):
        colv[pl.ds(k * 16, 16)] = jax.lax.shift_right_logical(
            suv[pl.ds(k * 16, 16)], 7)
    colv[pl.ds(B_PER_W, 16)] = jnp.full((16,), SENTINEL, jnp.int32)

    c_lo = colv[pl.ds(0, 16)][0]
    c_hi = colv[pl.ds(B_PER_W - 16, 16)][15]

    wchunks = [wv[pl.ds(q * 16, 16)] for q in range(EMBED // 16)]
    iot = lax.iota(jnp.int32, 16)
    ehq = [(iot + 16 * q) >> 3 for q in range(4)]
    elq = [(iot + 16 * q) & 7 for q in range(4)]
    eoffq = [iot + 16 * q for q in range(4)]

    def fire(c, b):
        @pl.when(c <= c_hi)
        def _():
            pltpu.async_copy(
                ut3.at[:, :, pl.ds(c * CHUNK, CHUNK)], sbuf.at[b], sems.at[b])

    def drain(c, b):
        @pl.when(c <= c_hi)
        def _():
            pltpu.make_async_copy(
                ut3.at[:, :, pl.ds(c * CHUNK, CHUNK)], sbuf.at[b],
                sems.at[b]).wait()

    def extract(c, b, carry):
        def cond(carry):
            jp, col_j = carry
            return jnp.logical_and(jp < B_PER_W, col_j == c)

        def wbody(carry):
            jp, _ = carry
            jpv = jnp.full((16,), jp, jnp.int32)
            su_j = plsc.load_gather(suv, [jpv])
            liv = su_j & 127
            fbase = jpv << 6
            for q in range(4):
                val = plsc.load_gather(sbuf, [jnp.full((16,), b, jnp.int32),
                                              ehq[q], elq[q], liv])
                flat = fbase + eoffq[q]
                plsc.store_scatter(
                    ustage, [jax.lax.shift_right_logical(flat, 7), flat & 127],
                    val * wchunks[q])
            jp1 = jp + 1
            col1 = plsc.load_gather(colv, [jnp.full((16,), jp1, jnp.int32)])[0]
            return (jp1, col1)

        return lax.while_loop(cond, wbody, carry)

    for r in range(RING):
        fire(c_lo + r, r)
    nsteps = jax.lax.shift_right_logical(c_hi - c_lo + RING, 2)

    def ring_body(t, carry):
        c0 = c_lo + RING * t
        for r in range(RING):
            drain(c0 + r, r)
            carry = extract(c0 + r, r, carry)
            fire(c0 + r + RING, r)
        return carry

    col0 = colv[pl.ds(0, 16)][0]
    lax.fori_loop(0, nsteps, ring_body, (jnp.int32(0), col0))

    pltpu.sync_copy(ustage, u1d_hbm.at[pl.ds(wid * (B_PER_W // 2),
                                             B_PER_W // 2)])


def _k2_join(u1d_hbm, inv_hbm, pids_hbm, product_t2, wb_hbm, out_hbm,
             iidx, pidx, iphys, pphys, ubuf, pbuf, wv, outv, sems):
    wid = lax.axis_index("s") * NC + lax.axis_index("c")
    base = wid * B_PER_W

    pltpu.sync_copy(inv_hbm.at[pl.ds(base, B_PER_W)], iidx)
    pltpu.sync_copy(pids_hbm.at[pl.ds(base, B_PER_W)], pidx)
    pltpu.sync_copy(wb_hbm, wv)

    for k in range(B_PER_W // 16):
        sl = pl.ds(k * 16, 16)
        iphys[sl] = jax.lax.shift_right_logical(iidx[sl], 1)
        pphys[sl] = jax.lax.shift_right_logical(pidx[sl], 1)

    def fire(j):
        b = j % 2
        return (
            pltpu.async_copy(
                u1d_hbm.at[iphys.at[pl.ds(j * CHUNK, CHUNK)]], ubuf.at[b],
                sems.at[b]),
            pltpu.async_copy(
                product_t2.at[pphys.at[pl.ds(j * CHUNK, CHUNK)]], pbuf.at[b],
                sems.at[b]),
        )

    lane = lax.iota(jnp.int32, 16)
    bias = wv[pl.ds(EMBED, 16)][0]

    inflight = fire(0)
    for j in range(NCHUNK):
        for c in inflight:
            c.wait()
        if j + 1 < NCHUNK:
            inflight = fire(j + 1)
        urows = ubuf.at[j % 2]
        prows = pbuf.at[j % 2]

        def body(g, carry):
            k = j * CHUNK + g * 16
            rows = lane + g * 16
            ucol0 = (iidx[pl.ds(k, 16)] & 1) * EMBED
            pcol0 = (pidx[pl.ds(k, 16)] & 1) * EMBED
            acc = jnp.zeros((16,), jnp.float32)
            for e in range(EMBED):
                u = plsc.load_gather(urows, [rows, ucol0 + e])
                p = plsc.load_gather(prows, [rows, pcol0 + e])
                acc = acc + u * p
            outv[pl.ds(k, 16)] = acc + bias
            return carry

        lax.fori_loop(0, CHUNK // 16, body, 0)

    pltpu.sync_copy(outv, out_hbm.at[pl.ds(base, B_PER_W)])


@jax.jit
def kernel(user_ids, product_ids, user_table, product_table, fc_w, fc_b):
    uids = user_ids.astype(jnp.int32)
    pids = product_ids.astype(jnp.int32)
    iota = lax.iota(jnp.int32, BATCH)
    su, order = lax.sort_key_val(uids, iota)
    _, inv = lax.sort_key_val(order, iota)

    ut3 = user_table.T.reshape(8, 8, user_table.shape[0])
    product_t2 = product_table.reshape(product_table.shape[0] // 2, 2 * EMBED)

    wb = jnp.zeros((128,), jnp.float32)
    wb = wb.at[:EMBED].set(fc_w[0]).at[EMBED].set(fc_b[0])

    mesh = plsc.VectorSubcoreMesh(core_axis_name="c", subcore_axis_name="s")
    params = pltpu.CompilerParams(needs_layout_passes=False)

    k1 = pl.kernel(
        _k1_user,
        out_type=jax.ShapeDtypeStruct((BATCH // 2, 2 * EMBED), jnp.float32),
        mesh=mesh, compiler_params=params,
        scratch_types=[
            pltpu.VMEM((B_PER_W,), jnp.int32),
            pltpu.VMEM((B_PER_W + 16,), jnp.int32),
            pltpu.VMEM((RING, 8, 8, CHUNK), jnp.float32),
            pltpu.VMEM((B_PER_W // 2, 2 * EMBED), jnp.float32),
            pltpu.VMEM((128,), jnp.float32),
            pltpu.SemaphoreType.DMA((RING,)),
        ],
    )
    u1d = k1(su, ut3, wb)

    k2 = pl.kernel(
        _k2_join,
        out_type=jax.ShapeDtypeStruct((BATCH,), jnp.float32),
        mesh=mesh, compiler_params=params,
        scratch_types=[
            pltpu.VMEM((B_PER_W,), jnp.int32),
            pltpu.VMEM((B_PER_W,), jnp.int32),
            pltpu.VMEM((B_PER_W,), jnp.int32),
            pltpu.VMEM((B_PER_W,), jnp.int32),
            pltpu.VMEM((2, CHUNK, 2 * EMBED), jnp.float32),
            pltpu.VMEM((2, CHUNK, 2 * EMBED), jnp.float32),
            pltpu.VMEM((128,), jnp.float32),
            pltpu.VMEM((B_PER_W,), jnp.float32),
            pltpu.SemaphoreType.DMA((2,)),
        ],
    )
    return k2(u1d, inv, pids, product_t2, wb)


# speedup vs baseline: 3.3473x; 1.0250x over previous
import jax
import jax.numpy as jnp
from jax import lax
from jax.experimental import pallas as pl
from jax.experimental.pallas import tpu as pltpu
from jax.experimental.pallas import tpu_sc as plsc

BATCH = 16384
EMBED = 64
NC = 2
NS = 16
NW = NC * NS
B_PER_W = BATCH // NW
CHUNK = 128
NCHUNK = B_PER_W // CHUNK
RING = 8
CH2 = 64
NCH2 = B_PER_W // CH2
RING2 = 4
SENTINEL = 0x7FFFFF


def _k1_user(su_hbm, ut3, wb_hbm, u1d_hbm, suv, colv, sbuf, ustage, wv, sems):
    wid = lax.axis_index("s") * NC + lax.axis_index("c")
    base = wid * B_PER_W

    pltpu.sync_copy(su_hbm.at[pl.ds(base, B_PER_W)], suv)
    pltpu.sync_copy(wb_hbm, wv)

    for k in range(B_PER_W // 16):
        colv[pl.ds(k * 16, 16)] = jax.lax.shift_right_logical(
            suv[pl.ds(k * 16, 16)], 7)
    colv[pl.ds(B_PER_W, 16)] = jnp.full((16,), SENTINEL, jnp.int32)

    c_lo = colv[pl.ds(0, 16)][0]
    c_hi = colv[pl.ds(B_PER_W - 16, 16)][15]

    wchunks = [wv[pl.ds(q * 16, 16)] for q in range(EMBED // 16)]
    iot = lax.iota(jnp.int32, 16)
    ehq = [(iot + 16 * q) >> 3 for q in range(4)]
    elq = [(iot + 16 * q) & 7 for q in range(4)]
    eoffq = [iot + 16 * q for q in range(4)]

    def fire(c, b):
        @pl.when(c <= c_hi)
        def _():
            pltpu.async_copy(
                ut3.at[:, :, pl.ds(c * CHUNK, CHUNK)], sbuf.at[b], sems.at[b])

    def drain(c, b):
        @pl.when(c <= c_hi)
        def _():
            pltpu.make_async_copy(
                ut3.at[:, :, pl.ds(c * CHUNK, CHUNK)], sbuf.at[b],
                sems.at[b]).wait()

    def extract(c, b, carry):
        def cond(carry):
            jp, col_j = carry
            return jnp.logical_and(jp < B_PER_W, col_j == c)

        def wbody(carry):
            jp, _ = carry
            jpv = jnp.full((16,), jp, jnp.int32)
            su_j = plsc.load_gather(suv, [jpv])
            liv = su_j & 127
            fbase = jpv << 6
            for q in range(4):
                val = plsc.load_gather(sbuf, [jnp.full((16,), b, jnp.int32),
                                              ehq[q], elq[q], liv])
                flat = fbase + eoffq[q]
                plsc.store_scatter(
                    ustage, [jax.lax.shift_right_logical(flat, 7), flat & 127],
                    val * wchunks[q])
            jp1 = jp + 1
            col1 = plsc.load_gather(colv, [jnp.full((16,), jp1, jnp.int32)])[0]
            return (jp1, col1)

        return lax.while_loop(cond, wbody, carry)

    for r in range(RING):
        fire(c_lo + r, r)
    nsteps = (c_hi - c_lo + RING) // RING

    def ring_body(t, carry):
        c0 = c_lo + RING * t
        for r in range(RING):
            drain(c0 + r, r)
            carry = extract(c0 + r, r, carry)
            fire(c0 + r + RING, r)
        return carry

    col0 = colv[pl.ds(0, 16)][0]
    lax.fori_loop(0, nsteps, ring_body, (jnp.int32(0), col0))

    pltpu.sync_copy(ustage, u1d_hbm.at[pl.ds(wid * (B_PER_W // 2),
                                             B_PER_W // 2)])


def _k2_join(u1d_hbm, inv_hbm, pids_hbm, product_t2, wb_hbm, out_hbm,
             iidx, pidx, iphys, pphys, ubuf, pbuf, wv, outv, sems):
    wid = lax.axis_index("s") * NC + lax.axis_index("c")
    base = wid * B_PER_W

    pltpu.sync_copy(inv_hbm.at[pl.ds(base, B_PER_W)], iidx)
    pltpu.sync_copy(pids_hbm.at[pl.ds(base, B_PER_W)], pidx)
    pltpu.sync_copy(wb_hbm, wv)

    for k in range(B_PER_W // 16):
        sl = pl.ds(k * 16, 16)
        iphys[sl] = jax.lax.shift_right_logical(iidx[sl], 1)
        pphys[sl] = jax.lax.shift_right_logical(pidx[sl], 1)

    def fire(j):
        b = j % RING2
        return (
            pltpu.async_copy(
                u1d_hbm.at[iphys.at[pl.ds(j * CH2, CH2)]], ubuf.at[b],
                sems.at[b]),
            pltpu.async_copy(
                product_t2.at[pphys.at[pl.ds(j * CH2, CH2)]], pbuf.at[b],
                sems.at[b]),
        )

    lane = lax.iota(jnp.int32, 16)
    bias = wv[pl.ds(EMBED, 16)][0]

    inflight = {j: fire(j) for j in range(RING2 - 1)}
    for j in range(NCH2):
        if j + RING2 - 1 < NCH2:
            inflight[j + RING2 - 1] = fire(j + RING2 - 1)
        for c in inflight.pop(j):
            c.wait()
        urows = ubuf.at[j % RING2]
        prows = pbuf.at[j % RING2]

        def body(g, carry):
            k = j * CH2 + g * 16
            rows = lane + g * 16
            ucol0 = (iidx[pl.ds(k, 16)] & 1) * EMBED
            pcol0 = (pidx[pl.ds(k, 16)] & 1) * EMBED
            acc = jnp.zeros((16,), jnp.float32)
            for e in range(EMBED):
                u = plsc.load_gather(urows, [rows, ucol0 + e])
                p = plsc.load_gather(prows, [rows, pcol0 + e])
                acc = acc + u * p
            outv[pl.ds(k, 16)] = acc + bias
            return carry

        lax.fori_loop(0, CH2 // 16, body, 0)

    pltpu.sync_copy(outv, out_hbm.at[pl.ds(base, B_PER_W)])


@jax.jit
def kernel(user_ids, product_ids, user_table, product_table, fc_w, fc_b):
    uids = user_ids.astype(jnp.int32)
    pids = product_ids.astype(jnp.int32)
    iota = lax.iota(jnp.int32, BATCH)
    su, order = lax.sort_key_val(uids, iota)
    _, inv = lax.sort_key_val(order, iota)

    ut3 = user_table.T.reshape(8, 8, user_table.shape[0])
    product_t2 = product_table.reshape(product_table.shape[0] // 2, 2 * EMBED)

    wb = jnp.zeros((128,), jnp.float32)
    wb = wb.at[:EMBED].set(fc_w[0]).at[EMBED].set(fc_b[0])

    mesh = plsc.VectorSubcoreMesh(core_axis_name="c", subcore_axis_name="s")
    params = pltpu.CompilerParams(needs_layout_passes=False)

    k1 = pl.kernel(
        _k1_user,
        out_type=jax.ShapeDtypeStruct((BATCH // 2, 2 * EMBED), jnp.float32),
        mesh=mesh, compiler_params=params,
        scratch_types=[
            pltpu.VMEM((B_PER_W,), jnp.int32),
            pltpu.VMEM((B_PER_W + 16,), jnp.int32),
            pltpu.VMEM((RING, 8, 8, CHUNK), jnp.float32),
            pltpu.VMEM((B_PER_W // 2, 2 * EMBED), jnp.float32),
            pltpu.VMEM((128,), jnp.float32),
            pltpu.SemaphoreType.DMA((RING,)),
        ],
    )
    u1d = k1(su, ut3, wb)

    k2 = pl.kernel(
        _k2_join,
        out_type=jax.ShapeDtypeStruct((BATCH,), jnp.float32),
        mesh=mesh, compiler_params=params,
        scratch_types=[
            pltpu.VMEM((B_PER_W,), jnp.int32),
            pltpu.VMEM((B_PER_W,), jnp.int32),
            pltpu.VMEM((B_PER_W,), jnp.int32),
            pltpu.VMEM((B_PER_W,), jnp.int32),
            pltpu.VMEM((RING2, CH2, 2 * EMBED), jnp.float32),
            pltpu.VMEM((RING2, CH2, 2 * EMBED), jnp.float32),
            pltpu.VMEM((128,), jnp.float32),
            pltpu.VMEM((B_PER_W,), jnp.float32),
            pltpu.SemaphoreType.DMA((RING2,)),
        ],
    )
    return k2(u1d, inv, pids, product_t2, wb)


# speedup vs baseline: 3.4101x; 1.0188x over previous
import jax
import jax.numpy as jnp
from jax import lax
from jax.experimental import pallas as pl
from jax.experimental.pallas import tpu as pltpu
from jax.experimental.pallas import tpu_sc as plsc

BATCH = 16384
EMBED = 64
NC = 2
NS = 16
NW = NC * NS
B_PER_W = BATCH // NW
CHUNK = 128
NCHUNK = B_PER_W // CHUNK
RING = 8
CH2 = 256
NCH2 = B_PER_W // CH2
RING2 = 1
SENTINEL = 0x7FFFFF


def _k1_user(su_hbm, ut3, wb_hbm, u1d_hbm, suv, colv, sbuf, ustage, wv, sems):
    wid = lax.axis_index("s") * NC + lax.axis_index("c")
    base = wid * B_PER_W

    pltpu.sync_copy(su_hbm.at[pl.ds(base, B_PER_W)], suv)
    pltpu.sync_copy(wb_hbm, wv)

    for k in range(B_PER_W // 16):
        colv[pl.ds(k * 16, 16)] = jax.lax.shift_right_logical(
            suv[pl.ds(k * 16, 16)], 7)
    colv[pl.ds(B_PER_W, 16)] = jnp.full((16,), SENTINEL, jnp.int32)

    c_lo = colv[pl.ds(0, 16)][0]
    c_hi = colv[pl.ds(B_PER_W - 16, 16)][15]

    wchunks = [wv[pl.ds(q * 16, 16)] for q in range(EMBED // 16)]
    iot = lax.iota(jnp.int32, 16)
    ehq = [(iot + 16 * q) >> 3 for q in range(4)]
    elq = [(iot + 16 * q) & 7 for q in range(4)]
    eoffq = [iot + 16 * q for q in range(4)]

    def fire(c, b):
        @pl.when(c <= c_hi)
        def _():
            pltpu.async_copy(
                ut3.at[:, :, pl.ds(c * CHUNK, CHUNK)], sbuf.at[b], sems.at[b])

    def drain(c, b):
        @pl.when(c <= c_hi)
        def _():
            pltpu.make_async_copy(
                ut3.at[:, :, pl.ds(c * CHUNK, CHUNK)], sbuf.at[b],
                sems.at[b]).wait()

    def extract(c, b, carry):
        def cond(carry):
            jp, col_j = carry
            return jnp.logical_and(jp < B_PER_W, col_j == c)

        def wbody(carry):
            jp, _ = carry
            jpv = jnp.full((16,), jp, jnp.int32)
            su_j = plsc.load_gather(suv, [jpv])
            liv = su_j & 127
            fbase = jpv << 6
            for q in range(4):
                val = plsc.load_gather(sbuf, [jnp.full((16,), b, jnp.int32),
                                              ehq[q], elq[q], liv])
                flat = fbase + eoffq[q]
                plsc.store_scatter(
                    ustage, [jax.lax.shift_right_logical(flat, 7), flat & 127],
                    val * wchunks[q])
            jp1 = jp + 1
            col1 = plsc.load_gather(colv, [jnp.full((16,), jp1, jnp.int32)])[0]
            return (jp1, col1)

        return lax.while_loop(cond, wbody, carry)

    for r in range(RING):
        fire(c_lo + r, r)
    nsteps = (c_hi - c_lo + RING) // RING

    def ring_body(t, carry):
        c0 = c_lo + RING * t
        for r in range(RING):
            drain(c0 + r, r)
            carry = extract(c0 + r, r, carry)
            fire(c0 + r + RING, r)
        return carry

    col0 = colv[pl.ds(0, 16)][0]
    lax.fori_loop(0, nsteps, ring_body, (jnp.int32(0), col0))

    pltpu.sync_copy(ustage, u1d_hbm.at[pl.ds(wid * (B_PER_W // 2),
                                             B_PER_W // 2)])


def _k2_join(u1d_hbm, inv_hbm, pids_hbm, product_t2, wb_hbm, out_hbm,
             iidx, pidx, iphys, pphys, ubuf, pbuf, wv, outv, sems):
    wid = lax.axis_index("s") * NC + lax.axis_index("c")
    base = wid * B_PER_W

    pltpu.sync_copy(inv_hbm.at[pl.ds(base, B_PER_W)], iidx)
    pltpu.sync_copy(pids_hbm.at[pl.ds(base, B_PER_W)], pidx)
    pltpu.sync_copy(wb_hbm, wv)

    for k in range(B_PER_W // 16):
        sl = pl.ds(k * 16, 16)
        iphys[sl] = jax.lax.shift_right_logical(iidx[sl], 1)
        pphys[sl] = jax.lax.shift_right_logical(pidx[sl], 1)

    def fire(j):
        return (
            pltpu.async_copy(
                u1d_hbm.at[iphys.at[pl.ds(j * CH2, CH2)]], ubuf,
                sems.at[0]),
            pltpu.async_copy(
                product_t2.at[pphys.at[pl.ds(j * CH2, CH2)]], pbuf,
                sems.at[1]),
        )

    lane = lax.iota(jnp.int32, 16)
    bias = wv[pl.ds(EMBED, 16)][0]

    for j in range(NCH2):
        inflight = fire(j)
        for c in inflight:
            c.wait()
        urows = ubuf
        prows = pbuf

        def body(g, carry):
            k = j * CH2 + g * 16
            rows = lane + g * 16
            ucol0 = (iidx[pl.ds(k, 16)] & 1) * EMBED
            pcol0 = (pidx[pl.ds(k, 16)] & 1) * EMBED
            acc = jnp.zeros((16,), jnp.float32)
            for e in range(EMBED):
                u = plsc.load_gather(urows, [rows, ucol0 + e])
                p = plsc.load_gather(prows, [rows, pcol0 + e])
                acc = acc + u * p
            outv[pl.ds(k, 16)] = acc + bias
            return carry

        lax.fori_loop(0, CH2 // 16, body, 0)

    pltpu.sync_copy(outv, out_hbm.at[pl.ds(base, B_PER_W)])


@jax.jit
def kernel(user_ids, product_ids, user_table, product_table, fc_w, fc_b):
    uids = user_ids.astype(jnp.int32)
    pids = product_ids.astype(jnp.int32)
    iota = lax.iota(jnp.int32, BATCH)
    su, order = lax.sort_key_val(uids, iota)
    _, inv = lax.sort_key_val(order, iota)

    ut3 = user_table.T.reshape(8, 8, user_table.shape[0])
    product_t2 = product_table.reshape(product_table.shape[0] // 2, 2 * EMBED)

    wb = jnp.zeros((128,), jnp.float32)
    wb = wb.at[:EMBED].set(fc_w[0]).at[EMBED].set(fc_b[0])

    mesh = plsc.VectorSubcoreMesh(core_axis_name="c", subcore_axis_name="s")
    params = pltpu.CompilerParams(needs_layout_passes=False)

    k1 = pl.kernel(
        _k1_user,
        out_type=jax.ShapeDtypeStruct((BATCH // 2, 2 * EMBED), jnp.float32),
        mesh=mesh, compiler_params=params,
        scratch_types=[
            pltpu.VMEM((B_PER_W,), jnp.int32),
            pltpu.VMEM((B_PER_W + 16,), jnp.int32),
            pltpu.VMEM((RING, 8, 8, CHUNK), jnp.float32),
            pltpu.VMEM((B_PER_W // 2, 2 * EMBED), jnp.float32),
            pltpu.VMEM((128,), jnp.float32),
            pltpu.SemaphoreType.DMA((RING,)),
        ],
    )
    u1d = k1(su, ut3, wb)

    k2 = pl.kernel(
        _k2_join,
        out_type=jax.ShapeDtypeStruct((BATCH,), jnp.float32),
        mesh=mesh, compiler_params=params,
        scratch_types=[
            pltpu.VMEM((B_PER_W,), jnp.int32),
            pltpu.VMEM((B_PER_W,), jnp.int32),
            pltpu.VMEM((B_PER_W,), jnp.int32),
            pltpu.VMEM((B_PER_W,), jnp.int32),
            pltpu.VMEM((CH2, 2 * EMBED), jnp.float32),
            pltpu.VMEM((CH2, 2 * EMBED), jnp.float32),
            pltpu.VMEM((128,), jnp.float32),
            pltpu.VMEM((B_PER_W,), jnp.float32),
            pltpu.SemaphoreType.DMA((RING2,)),
        ],
    )
    return k2(u1d, inv, pids, product_t2, wb)


# speedup vs baseline: 3.6306x; 1.0647x over previous
import jax
import jax.numpy as jnp
from jax import lax
from jax.experimental import pallas as pl
from jax.experimental.pallas import tpu as pltpu
from jax.experimental.pallas import tpu_sc as plsc

BATCH = 16384
EMBED = 64
NC = 2
NS = 16
NW = NC * NS
B_PER_W = BATCH // NW
CHUNK = 128
NCHUNK = B_PER_W // CHUNK
RING = 8
CH2 = 256
NCH2 = B_PER_W // CH2
RING2 = 1
SENTINEL = 0x7FFFFF


def _k1_user(su_hbm, ut3, wb_hbm, u1d_hbm, suv, colv, ucols, sbuf, ustage,
             wv, sems):
    wid = lax.axis_index("s") * NC + lax.axis_index("c")
    base = wid * B_PER_W

    pltpu.sync_copy(su_hbm.at[pl.ds(base, B_PER_W)], suv)
    pltpu.sync_copy(wb_hbm, wv)

    iot = lax.iota(jnp.int32, 16)

    for k in range(B_PER_W // 16):
        colv[pl.ds(k * 16, 16)] = jax.lax.shift_right_logical(
            suv[pl.ds(k * 16, 16)], 7)
    colv[pl.ds(B_PER_W, 16)] = jnp.full((16,), SENTINEL, jnp.int32)

    nu = jnp.int32(0)
    for k in range(B_PER_W // 16):
        cur = colv[pl.ds(k * 16, 16)]
        prev = plsc.load_gather(colv, [jnp.maximum(iot + (k * 16 - 1), 0)])
        m = cur != prev
        if k == 0:
            m = jnp.logical_or(m, iot == 0)
        ranks = plsc.cumsum(m.astype(jnp.int32))
        plsc.store_scatter(ucols, [ranks + (nu - 1)], cur, mask=m)
        nu = nu + ranks[15]

    wchunks = [wv[pl.ds(q * 16, 16)] for q in range(EMBED // 16)]
    ehq = [(iot + 16 * q) >> 3 for q in range(4)]
    elq = [(iot + 16 * q) & 7 for q in range(4)]
    eoffq = [iot + 16 * q for q in range(4)]

    def ucol(m):
        return plsc.load_gather(ucols, [jnp.full((16,), m, jnp.int32)])[0]

    def fire(m, b):
        @pl.when(m < nu)
        def _():
            c = ucol(m)
            pltpu.async_copy(
                ut3.at[:, :, pl.ds(c * CHUNK, CHUNK)], sbuf.at[b], sems.at[b])

    def drain(m, b):
        @pl.when(m < nu)
        def _():
            pltpu.make_async_copy(
                ut3.at[:, :, pl.ds(0, CHUNK)], sbuf.at[b], sems.at[b]).wait()

    def extract(m, b, carry):
        c = ucol(m)
        def cond(carry):
            jp, col_j = carry
            return jnp.logical_and(jp < B_PER_W, col_j == c)

        def wbody(carry):
            jp, _ = carry
            jpv = jnp.full((16,), jp, jnp.int32)
            su_j = plsc.load_gather(suv, [jpv])
            liv = su_j & 127
            fbase = jpv << 6
            for q in range(4):
                val = plsc.load_gather(sbuf, [jnp.full((16,), b, jnp.int32),
                                              ehq[q], elq[q], liv])
                flat = fbase + eoffq[q]
                plsc.store_scatter(
                    ustage, [jax.lax.shift_right_logical(flat, 7), flat & 127],
                    val * wchunks[q])
            jp1 = jp + 1
            col1 = plsc.load_gather(colv, [jnp.full((16,), jp1, jnp.int32)])[0]
            return (jp1, col1)

        return lax.while_loop(cond, wbody, carry)

    for r in range(RING):
        fire(jnp.int32(r), r)
    nsteps = (nu + RING - 1) // RING

    def ring_body(t, carry):
        m0 = RING * t
        for r in range(RING):
            drain(m0 + r, r)
            carry = extract(m0 + r, r, carry)
            fire(m0 + r + RING, r)
        return carry

    col0 = colv[pl.ds(0, 16)][0]
    lax.fori_loop(0, nsteps, ring_body, (jnp.int32(0), col0))

    pltpu.sync_copy(ustage, u1d_hbm.at[pl.ds(wid * (B_PER_W // 2),
                                             B_PER_W // 2)])


def _k2_join(u1d_hbm, inv_hbm, pids_hbm, product_t2, wb_hbm, out_hbm,
             iidx, pidx, iphys, pphys, ubuf, pbuf, wv, outv, sems):
    wid = lax.axis_index("s") * NC + lax.axis_index("c")
    base = wid * B_PER_W

    pltpu.sync_copy(inv_hbm.at[pl.ds(base, B_PER_W)], iidx)
    pltpu.sync_copy(pids_hbm.at[pl.ds(base, B_PER_W)], pidx)
    pltpu.sync_copy(wb_hbm, wv)

    for k in range(B_PER_W // 16):
        sl = pl.ds(k * 16, 16)
        iphys[sl] = jax.lax.shift_right_logical(iidx[sl], 1)
        pphys[sl] = jax.lax.shift_right_logical(pidx[sl], 1)

    def fire(j):
        return (
            pltpu.async_copy(
                u1d_hbm.at[iphys.at[pl.ds(j * CH2, CH2)]], ubuf,
                sems.at[0]),
            pltpu.async_copy(
                product_t2.at[pphys.at[pl.ds(j * CH2, CH2)]], pbuf,
                sems.at[1]),
        )

    lane = lax.iota(jnp.int32, 16)
    bias = wv[pl.ds(EMBED, 16)][0]

    for j in range(NCH2):
        inflight = fire(j)
        for c in inflight:
            c.wait()
        urows = ubuf
        prows = pbuf

        def body(g, carry):
            k = j * CH2 + g * 16
            rows = lane + g * 16
            ucol0 = (iidx[pl.ds(k, 16)] & 1) * EMBED
            pcol0 = (pidx[pl.ds(k, 16)] & 1) * EMBED
            acc = jnp.zeros((16,), jnp.float32)
            for e in range(EMBED):
                u = plsc.load_gather(urows, [rows, ucol0 + e])
                p = plsc.load_gather(prows, [rows, pcol0 + e])
                acc = acc + u * p
            outv[pl.ds(k, 16)] = acc + bias
            return carry

        lax.fori_loop(0, CH2 // 16, body, 0)

    pltpu.sync_copy(outv, out_hbm.at[pl.ds(base, B_PER_W)])


@jax.jit
def kernel(user_ids, product_ids, user_table, product_table, fc_w, fc_b):
    uids = user_ids.astype(jnp.int32)
    pids = product_ids.astype(jnp.int32)
    iota = lax.iota(jnp.int32, BATCH)
    su, order = lax.sort_key_val(uids, iota)
    _, inv = lax.sort_key_val(order, iota)

    ut3 = user_table.T.reshape(8, 8, user_table.shape[0])
    product_t2 = product_table.reshape(product_table.shape[0] // 2, 2 * EMBED)

    wb = jnp.zeros((128,), jnp.float32)
    wb = wb.at[:EMBED].set(fc_w[0]).at[EMBED].set(fc_b[0])

    mesh = plsc.VectorSubcoreMesh(core_axis_name="c", subcore_axis_name="s")
    params = pltpu.CompilerParams(needs_layout_passes=False)

    k1 = pl.kernel(
        _k1_user,
        out_type=jax.ShapeDtypeStruct((BATCH // 2, 2 * EMBED), jnp.float32),
        mesh=mesh, compiler_params=params,
        scratch_types=[
            pltpu.VMEM((B_PER_W,), jnp.int32),
            pltpu.VMEM((B_PER_W + 16,), jnp.int32),
            pltpu.VMEM((B_PER_W,), jnp.int32),
            pltpu.VMEM((RING, 8, 8, CHUNK), jnp.float32),
            pltpu.VMEM((B_PER_W // 2, 2 * EMBED), jnp.float32),
            pltpu.VMEM((128,), jnp.float32),
            pltpu.SemaphoreType.DMA((RING,)),
        ],
    )
    u1d = k1(su, ut3, wb)

    k2 = pl.kernel(
        _k2_join,
        out_type=jax.ShapeDtypeStruct((BATCH,), jnp.float32),
        mesh=mesh, compiler_params=params,
        scratch_types=[
            pltpu.VMEM((B_PER_W,), jnp.int32),
            pltpu.VMEM((B_PER_W,), jnp.int32),
            pltpu.VMEM((B_PER_W,), jnp.int32),
            pltpu.VMEM((B_PER_W,), jnp.int32),
            pltpu.VMEM((CH2, 2 * EMBED), jnp.float32),
            pltpu.VMEM((CH2, 2 * EMBED), jnp.float32),
            pltpu.VMEM((128,), jnp.float32),
            pltpu.VMEM((B_PER_W,), jnp.float32),
            pltpu.SemaphoreType.DMA((RING2,)),
        ],
    )
    return k2(u1d, inv, pids, product_t2, wb)
